# Initial kernel scaffold; baseline (speedup 1.0000x reference)
import jax
import jax.numpy as jnp
from jax.experimental import pallas as pl


def kernel(x, edge_index, batch, Wl1, Wr1, b1, Wl2, Wr2, b2, Wl3, Wr3, b3, Wfc, bfc):
    raise NotImplementedError("write your pallas kernel here")


# speedup vs baseline: 6.7482x; 6.7482x over previous
import functools

import jax
import jax.numpy as jnp
from jax import lax
from jax.experimental import pallas as pl
from jax.experimental.pallas import tpu as pltpu
from jax.experimental.pallas import tpu_sc as plsc

N = 50000
E = 800000
HID = 128
NCLS = 10

NC = 2
NS = 16
CH = 128
J = 40
N_PAD = 50176
STRIPE = N_PAD // NS
E_PAD = 819200
R = E_PAD // CH
RPT = R // NS
RPW = R // (NC * NS)
G = 4
GW = HID // G

M1 = 1024
M2 = 512
M3 = 512

_MESH = dict(core_axis_name="c", subcore_axis_name="s")


@functools.partial(
    pl.kernel,
    out_type=(
        jax.ShapeDtypeStruct((2 * N_PAD,), jnp.float32),
        jax.ShapeDtypeStruct((2 * N_PAD,), jnp.float32),
    ),
    mesh=plsc.VectorSubcoreMesh(**_MESH),
    compiler_params=pltpu.CompilerParams(use_tc_tiling_on_sc=False),
    scratch_types=[
        pltpu.VMEM_SHARED((N_PAD,), jnp.float32),
        pltpu.VMEM_SHARED((N_PAD,), jnp.float32),
        pltpu.VMEM_SHARED((N_PAD,), jnp.float32),
        pltpu.VMEM((J, CH), jnp.int32),
        pltpu.VMEM((J, CH), jnp.int32),
        pltpu.VMEM((CH,), jnp.float32),
        pltpu.VMEM((CH,), jnp.float32),
        pltpu.VMEM((STRIPE,), jnp.float32),
    ],
)
def _sc_scalar_agg(x_hbm, src_hbm, dst_hbm, agg_hbm, cnt_hbm,
                   x_sp, acc_sp, cnt_sp, sstage, dstage, vals, ones, zstripe):
    c = lax.axis_index("c")
    s = lax.axis_index("s")
    base = s * STRIPE

    def fill_ones(i, carry):
        ones[pl.ds(i * 16, 16)] = jnp.ones((16,), jnp.float32)
        return carry
    lax.fori_loop(0, CH // 16, fill_ones, 0)

    def fill_zero(i, carry):
        zstripe[pl.ds(i * 16, 16)] = jnp.zeros((16,), jnp.float32)
        return carry
    lax.fori_loop(0, STRIPE // 16, fill_zero, 0)

    pltpu.sync_copy(zstripe, acc_sp.at[pl.ds(base, STRIPE)])
    pltpu.sync_copy(zstripe, cnt_sp.at[pl.ds(base, STRIPE)])
    pltpu.sync_copy(x_hbm.at[pl.ds(base, STRIPE)], zstripe)
    pltpu.sync_copy(zstripe, x_sp.at[pl.ds(base, STRIPE)])
    plsc.subcore_barrier()

    wid = c * NS + s

    def macro(m, carry):
        row0 = wid * RPW + m * J
        pltpu.sync_copy(src_hbm.at[pl.ds(row0, J)], sstage)
        pltpu.sync_copy(dst_hbm.at[pl.ds(row0, J)], dstage)

        def inner(j, inner_carry):
            pltpu.sync_copy(x_sp.at[sstage.at[j]], vals)
            pltpu.sync_copy(vals, acc_sp.at[dstage.at[j]], add=True)
            pltpu.sync_copy(ones, cnt_sp.at[dstage.at[j]], add=True)
            return inner_carry
        lax.fori_loop(0, J, inner, carry)
        return carry
    lax.fori_loop(0, RPW // J, macro, 0)

    plsc.subcore_barrier()
    out_off = c * N_PAD + base
    pltpu.sync_copy(acc_sp.at[pl.ds(base, STRIPE)], zstripe)
    pltpu.sync_copy(zstripe, agg_hbm.at[pl.ds(out_off, STRIPE)])
    pltpu.sync_copy(cnt_sp.at[pl.ds(base, STRIPE)], zstripe)
    pltpu.sync_copy(zstripe, cnt_hbm.at[pl.ds(out_off, STRIPE)])


@functools.partial(
    pl.kernel,
    out_type=tuple(
        jax.ShapeDtypeStruct((N_PAD, GW), jnp.float32) for _ in range(G)),
    mesh=plsc.VectorSubcoreMesh(**_MESH),
    compiler_params=pltpu.CompilerParams(use_tc_tiling_on_sc=False),
    scratch_types=[
        pltpu.VMEM_SHARED((N_PAD, GW), jnp.float32),
        pltpu.VMEM((J, CH), jnp.int32),
        pltpu.VMEM((J, CH), jnp.int32),
        pltpu.VMEM((CH, GW), jnp.float32),
        pltpu.VMEM((64, GW), jnp.float32),
        pltpu.VMEM((64, GW), jnp.float32),
    ],
)
def _sc_seg_sum(t0, t1, t2, t3, src_hbm, dst_hbm, o0, o1, o2, o3,
                acc_sp, sstage, dstage, rows, zblk, rows2):
    c = lax.axis_index("c")
    s = lax.axis_index("s")
    base = s * STRIPE
    tables = (t0, t1, t2, t3)
    outs = (o0, o1, o2, o3)

    def fill_zero(i, carry):
        r = i // (GW // 16)
        k = i % (GW // 16)
        zblk[r, pl.ds(k * 16, 16)] = jnp.zeros((16,), jnp.float32)
        return carry
    lax.fori_loop(0, 64 * (GW // 16), fill_zero, 0)

    for g in range(G):
        @pl.when(c == g // 2)
        def _process(g=g):
            table = tables[g]
            out = outs[g]

            def zero_stripe(i, carry):
                pltpu.sync_copy(zblk, acc_sp.at[pl.ds(base + i * 64, 64)])
                return carry
            lax.fori_loop(0, STRIPE // 64, zero_stripe, 0)
            plsc.subcore_barrier()

            def macro(m, carry):
                row0 = s * RPT + m * J
                pltpu.sync_copy(src_hbm.at[pl.ds(row0, J)], sstage)
                pltpu.sync_copy(dst_hbm.at[pl.ds(row0, J)], dstage)

                def inner(j, inner_carry):
                    pltpu.sync_copy(table.at[sstage.at[j]], rows)
                    pltpu.sync_copy(rows, acc_sp.at[dstage.at[j]], add=True)
                    return inner_carry
                lax.fori_loop(0, J, inner, carry)
                return carry
            lax.fori_loop(0, RPT // J, macro, 0)

            plsc.subcore_barrier()

            def wout(i, carry):
                pltpu.sync_copy(acc_sp.at[pl.ds(base + i * 64, 64)], rows2)
                pltpu.sync_copy(rows2, out.at[pl.ds(base + i * 64, 64)])
                return carry
            lax.fori_loop(0, STRIPE // 64, wout, 0)


def _tc1_body(aggp_ref, cntp_ref, x_ref, wl1_ref, wr1_ref, b1_ref,
              t0, t1, t2, t3, inv_ref):
    sa = aggp_ref[0] + aggp_ref[1]
    cn = cntp_ref[0] + cntp_ref[1]
    inv = 1.0 / jnp.maximum(cn, 1.0)
    inv_ref[...] = inv
    a = sa * inv
    h = jnp.maximum(a * wl1_ref[...] + x_ref[...] * wr1_ref[...] + b1_ref[...],
                    0.0)
    for g, tref in enumerate((t0, t1, t2, t3)):
        tref[...] = h[:, g * GW:(g + 1) * GW]


def _tc2_body(a0, a1, a2, a3, inv_ref, t0, t1, t2, t3,
              wl2, wr2, b2_ref, wl3, wr3,
              p0, p1, p2, p3, r_ref):
    agg = jnp.concatenate([a0[...], a1[...], a2[...], a3[...]], axis=1)
    agg = agg * inv_ref[...]
    h1 = jnp.concatenate([t0[...], t1[...], t2[...], t3[...]], axis=1)
    h2 = jnp.maximum(
        jnp.dot(agg, wl2[...], preferred_element_type=jnp.float32)
        + jnp.dot(h1, wr2[...], preferred_element_type=jnp.float32)
        + b2_ref[...], 0.0)
    p = jnp.dot(h2, wl3[...], preferred_element_type=jnp.float32)
    r_ref[...] = jnp.dot(h2, wr3[...], preferred_element_type=jnp.float32)
    for g, pref in enumerate((p0, p1, p2, p3)):
        pref[...] = p[:, g * GW:(g + 1) * GW]


def _tc3_body(a0, a1, a2, a3, inv_ref, r_ref, b3_ref, wfc, bfc_ref,
              out_ref, acc):
    i = pl.program_id(0)

    @pl.when(i == 0)
    def _init():
        acc[...] = jnp.zeros_like(acc)

    agg = jnp.concatenate([a0[...], a1[...], a2[...], a3[...]], axis=1)
    agg = agg * inv_ref[...]
    h3 = jnp.maximum(agg + r_ref[...] + b3_ref[...], 0.0)
    rowid = i * M3 + lax.broadcasted_iota(jnp.int32, (M3, HID), 0)
    h3 = jnp.where(rowid < N, h3, 0.0)
    acc[...] += h3.reshape(M3 // 8, 8, HID).sum(axis=0)

    @pl.when(i == (N_PAD // M3) - 1)
    def _finish():
        pooled = acc[...].sum(axis=0, keepdims=True) * jnp.float32(1.0 / N)
        out_ref[...] = (jnp.dot(pooled, wfc[...],
                                preferred_element_type=jnp.float32)
                        + bfc_ref[...])


def _tc1(aggp, cntp, xp2, Wl1, Wr1, b1):
    grid = (N_PAD // M1,)
    return pl.pallas_call(
        _tc1_body,
        grid=grid,
        in_specs=[
            pl.BlockSpec((2, M1, 1), lambda i: (0, i, 0)),
            pl.BlockSpec((2, M1, 1), lambda i: (0, i, 0)),
            pl.BlockSpec((M1, 1), lambda i: (i, 0)),
            pl.BlockSpec((1, HID), lambda i: (0, 0)),
            pl.BlockSpec((1, HID), lambda i: (0, 0)),
            pl.BlockSpec((1, HID), lambda i: (0, 0)),
        ],
        out_specs=[pl.BlockSpec((M1, GW), lambda i: (i, 0)) for _ in range(G)]
        + [pl.BlockSpec((M1, 1), lambda i: (i, 0))],
        out_shape=[jax.ShapeDtypeStruct((N_PAD, GW), jnp.float32)
                   for _ in range(G)]
        + [jax.ShapeDtypeStruct((N_PAD, 1), jnp.float32)],
    )(aggp, cntp, xp2, Wl1, Wr1, b1.reshape(1, HID))


def _tc2(aggs, inv, tabs, Wl2, Wr2, b2, Wl3, Wr3):
    grid = (N_PAD // M2,)
    full = lambda shape: pl.BlockSpec(shape, lambda i: (0, 0))
    return pl.pallas_call(
        _tc2_body,
        grid=grid,
        in_specs=[pl.BlockSpec((M2, GW), lambda i: (i, 0)) for _ in range(G)]
        + [pl.BlockSpec((M2, 1), lambda i: (i, 0))]
        + [pl.BlockSpec((M2, GW), lambda i: (i, 0)) for _ in range(G)]
        + [full((HID, 2 * HID)), full((HID, 2 * HID)), full((1, 2 * HID)),
           full((2 * HID, HID)), full((2 * HID, HID))],
        out_specs=[pl.BlockSpec((M2, GW), lambda i: (i, 0)) for _ in range(G)]
        + [pl.BlockSpec((M2, HID), lambda i: (i, 0))],
        out_shape=[jax.ShapeDtypeStruct((N_PAD, GW), jnp.float32)
                   for _ in range(G)]
        + [jax.ShapeDtypeStruct((N_PAD, HID), jnp.float32)],
    )(*aggs, inv, *tabs, Wl2, Wr2, b2.reshape(1, 2 * HID), Wl3, Wr3)


def _tc3(aggs, inv, r, b3, Wfc, bfc):
    grid = (N_PAD // M3,)
    full = lambda shape: pl.BlockSpec(shape, lambda i: (0, 0))
    return pl.pallas_call(
        _tc3_body,
        grid=grid,
        in_specs=[pl.BlockSpec((M3, GW), lambda i: (i, 0)) for _ in range(G)]
        + [pl.BlockSpec((M3, 1), lambda i: (i, 0)),
           pl.BlockSpec((M3, HID), lambda i: (i, 0)),
           full((1, HID)), full((HID, NCLS)), full((1, NCLS))],
        out_specs=pl.BlockSpec((1, NCLS), lambda i: (0, 0)),
        out_shape=jax.ShapeDtypeStruct((1, NCLS), jnp.float32),
        scratch_shapes=[pltpu.VMEM((8, HID), jnp.float32)],
    )(*aggs, inv, r, b3.reshape(1, HID), Wfc, bfc.reshape(1, NCLS))


def kernel(x, edge_index, batch, Wl1, Wr1, b1, Wl2, Wr2, b2,
           Wl3, Wr3, b3, Wfc, bfc):
    src = edge_index[0].astype(jnp.int32)
    dst = edge_index[1].astype(jnp.int32)
    pad_e = E_PAD - E
    pad_src = (jnp.arange(pad_e, dtype=jnp.int32) * 97) % N
    src_p = jnp.concatenate([src, pad_src]).reshape(R, CH)
    dst_p = jnp.concatenate(
        [dst, jnp.full((pad_e,), N, jnp.int32)]).reshape(R, CH)
    xf = jnp.pad(x[:, 0], (0, N_PAD - N))

    agg1f, cnt1f = _sc_scalar_agg(xf, src_p, dst_p)
    aggp = agg1f.reshape(2, N_PAD, 1)
    cntp = cnt1f.reshape(2, N_PAD, 1)

    *tabs, inv = _tc1(aggp, cntp, xf.reshape(N_PAD, 1), Wl1, Wr1, b1)
    a2 = _sc_seg_sum(*tabs, src_p, dst_p)
    *ptabs, r = _tc2(a2, inv, tabs, Wl2, Wr2, b2, Wl3, Wr3)
    a3 = _sc_seg_sum(*ptabs, src_p, dst_p)
    return _tc3(a3, inv, r, b3, Wfc, bfc)
